# baseline (speedup 1.0000x reference)
import functools

import jax
import jax.numpy as jnp
import numpy as np
from jax import lax
from jax.experimental import pallas as pl
from jax.experimental.pallas import tpu as pltpu
from jax.experimental.pallas import tpu_sc as plsc

B, N, C = 2, 4096, 128
K = 32
NUM_KNOTS = 32
RADIUS = 0.2
NE = N * K
M = B * N


def _graph_jax():
    H = int(N ** 0.5)
    gy, gx = jnp.meshgrid(jnp.linspace(0.0, 1.0, H), jnp.linspace(0.0, 1.0, H),
                          indexing="ij")
    coords = jnp.stack([gy, gx], axis=-1).reshape(N, 2).astype(jnp.float32)
    diff = coords[:, None, :] - coords[None, :, :]
    dist = jnp.linalg.norm(diff, axis=-1)
    dm = jnp.where(dist <= RADIUS, dist, jnp.inf)
    _, jidx = jax.lax.top_k(-dm, K)
    ci = jnp.repeat(coords, K, axis=0)
    cj = coords[jidx.reshape(-1)]
    r = jnp.linalg.norm(ci - cj, axis=-1)
    xy = jnp.concatenate([ci, cj], axis=-1)
    return ci, xy, r, jidx.astype(jnp.int32)


def _build_consts():
    try:
        ci, xy, r, jidx = (np.asarray(a) for a in
                           jax.block_until_ready(jax.jit(_graph_jax)()))
    except Exception:
        H = int(N ** 0.5)
        lin = np.arange(H, dtype=np.float32) * (np.float32(1.0) / np.float32(H - 1))
        gy, gx = np.meshgrid(lin, lin, indexing="ij")
        coords = np.stack([gy, gx], axis=-1).reshape(N, 2).astype(np.float32)
        d = coords[:, None, :] - coords[None, :, :]
        dist = np.sqrt(d[..., 0] * d[..., 0] + d[..., 1] * d[..., 1]).astype(np.float32)
        dm = np.where(dist <= np.float32(RADIUS), dist, np.float32(np.inf))
        jidx = np.argsort(dm, axis=1, kind="stable")[:, :K].astype(np.int32)
        ci = np.repeat(coords, K, axis=0)
        cj = coords[jidx.reshape(-1)]
        dd = ci - cj
        r = np.sqrt(dd[:, 0] * dd[:, 0] + dd[:, 1] * dd[:, 1]).astype(np.float32)
        xy = np.concatenate([ci, cj], axis=-1).astype(np.float32)
    jidx = np.asarray(jidx, dtype=np.int32).reshape(N, K)
    eidx2 = np.ascontiguousarray(jidx.reshape(-1, 4 * K))
    return (np.asarray(ci), np.asarray(xy),
            np.asarray(r, dtype=np.float32).reshape(NE, 1),
            eidx2)


_CI_NP, _XY_NP, _R_NP, _EIDX_NP = _build_consts()


BLK_E = 4096
GRID1 = NE // BLK_E


def _softplus(v):
    return jnp.maximum(v, 0.0) + jnp.log1p(jnp.exp(-jnp.abs(v)))


def _stage1_body(xy_ref, ci_ref, r_ref, pw1_ref, pb1_ref, pw2_ref, pb2_ref,
                 hw1_ref, hb1_ref, hw2t_ref, hb2_ref, stab_ref,
                 psiphi_ref, sphi_ref, spsi_ref):
    xy = xy_ref[...]
    ci = ci_ref[...]
    r = r_ref[...]
    hh = hb1_ref[...] + ci[:, 0:1] * hw1_ref[0:1, :] + ci[:, 1:2] * hw1_ref[1:2, :]
    hh = jnp.maximum(hh, 0.0)
    hlin = jnp.sum(hh * hw2t_ref[...], axis=1, keepdims=True) + hb2_ref[...]
    h = _softplus(hlin)
    rs = jnp.clip(r / (h + 1e-06), 0.0, 1.0)
    idx = jnp.clip(jnp.floor(rs * (NUM_KNOTS - 1)), 0.0, float(NUM_KNOTS - 2))
    idx = idx.astype(jnp.int32)
    step = np.float32(1.0 / (NUM_KNOTS - 1))
    idxf = idx.astype(jnp.float32)
    t_k = idxf * step
    t_k1 = (idxf + 1.0) * step
    wr = (rs - t_k) / (t_k1 - t_k + 1e-08)
    oh = (idx == lax.broadcasted_iota(jnp.int32, (BLK_E, NUM_KNOTS), 1))
    ohf = oh.astype(jnp.float32)
    s_k = jnp.sum(ohf * stab_ref[0:1, :], axis=1, keepdims=True)
    s_k1 = jnp.sum(ohf * stab_ref[1:2, :], axis=1, keepdims=True)
    psi = (1.0 - wr) * s_k + wr * s_k1
    ph = pb1_ref[...]
    for d in range(4):
        ph = ph + xy[:, d:d + 1] * pw1_ref[d:d + 1, :]
    ph = jnp.maximum(ph, 0.0)
    phi = lax.dot_general(ph, pw2_ref[...], (((1,), (0,)), ((), ())),
                          precision=lax.Precision.HIGHEST,
                          preferred_element_type=jnp.float32) + pb2_ref[...]
    psiphi_ref[...] = (psi * phi).astype(jnp.bfloat16)

    @pl.when(pl.program_id(0) == 0)
    def _():
        sphi_ref[...] = jnp.zeros_like(sphi_ref)
        spsi_ref[...] = jnp.zeros_like(spsi_ref)

    sphi_ref[...] += jnp.sum(jnp.abs(phi), axis=0, keepdims=True)
    spsi_ref[...] += jnp.full((1, C), jnp.sum(jnp.abs(psi)), jnp.float32)


def _stage1(xy, ci, r, pw1, pb1, pw2, pb2, hw1, hb1, hw2t, hb2, stab):
    full = lambda s: pl.BlockSpec(s, lambda i: (0, 0))
    return pl.pallas_call(
        _stage1_body,
        grid=(GRID1,),
        in_specs=[
            pl.BlockSpec((BLK_E, 4), lambda i: (i, 0)),
            pl.BlockSpec((BLK_E, 2), lambda i: (i, 0)),
            pl.BlockSpec((BLK_E, 1), lambda i: (i, 0)),
            full((4, C)), full((1, C)), full((C, C)), full((1, C)),
            full((2, C)), full((1, C)), full((1, C)), full((1, 1)),
            full((2, NUM_KNOTS)),
        ],
        out_specs=[
            pl.BlockSpec((BLK_E, C), lambda i: (i, 0)),
            pl.BlockSpec((1, C), lambda i: (0, 0)),
            pl.BlockSpec((1, C), lambda i: (0, 0)),
        ],
        out_shape=[
            jax.ShapeDtypeStruct((NE, C), jnp.bfloat16),
            jax.ShapeDtypeStruct((1, C), jnp.float32),
            jax.ShapeDtypeStruct((1, C), jnp.float32),
        ],
    )(xy, ci, r, pw1, pb1, pw2, pb2, hw1, hb1, hw2t, hb2, stab)


NW = 32
G = 4
NCORES = 2
VS = C // 16


NPW = N // NW
NCH = NPW // G
EPC = G * K
CW = C // 2
WG = CW // 16


def _agg_body(x_hbm, w_hbm, eidx_hbm, out_hbm, idx_all,
              wb0, wb1, xa0, xa1, ob00, ob01, ob10, ob11,
              semi0, semi1, semo0, semo1):
    wid = lax.axis_index("s") * NCORES + lax.axis_index("c")
    n0 = wid * NPW
    pltpu.sync_copy(eidx_hbm.at[pl.ds(wid * NCH, NCH)], idx_all)

    wbufs = (wb0, wb1)
    xbufs = (xa0, xa1)
    obufs = ((ob00, ob01), (ob10, ob11))
    semis = (semi0, semi1)
    semos = (semo0, semo1)

    def in_copies(t, p):
        ib = n0 + t * G
        return (
            pltpu.make_async_copy(w_hbm.at[pl.ds(ib * K, EPC)], wbufs[p], semis[p]),
            pltpu.make_async_copy(x_hbm.at[idx_all.at[t]], xbufs[p], semis[p]),
        )

    def out_copies(t, p):
        ib = n0 + t * G
        return (
            pltpu.make_async_copy(obufs[p][0], out_hbm.at[pl.ds(ib, G)], semos[p]),
            pltpu.make_async_copy(obufs[p][1], out_hbm.at[pl.ds(N + ib, G)], semos[p]),
        )

    for c in in_copies(0, 0) + in_copies(1, 1):
        c.start()

    zero = tuple(jnp.zeros((16,), jnp.float32) for _ in range(VS))
    for t in range(NCH):
        p = t & 1
        if t >= 2:
            for c in out_copies(t - 2, p):
                c.wait()
        for c in in_copies(t, p):
            c.wait()
        wv_ref, xr = wbufs[p], xbufs[p]
        o0, o1 = obufs[p]

        def gloop(g, carry):
            def unpk(word):
                lo = lax.bitcast_convert_type(lax.shift_left(word, 16), jnp.float32)
                hi = lax.bitcast_convert_type(word & jnp.int32(-65536), jnp.float32)
                return lo, hi

            def edge(k, accs):
                a0, a1 = accs
                e = g * K + k
                n0_, n1_ = list(a0), list(a1)
                for vv in range(WG):
                    sl = pl.ds(16 * vv, 16)
                    wlo, whi = unpk(wv_ref[e, sl])
                    xlo0, xhi0 = unpk(xr[e, sl])
                    xlo1, xhi1 = unpk(xr[e, pl.ds(CW + 16 * vv, 16)])
                    n0_[2 * vv] = n0_[2 * vv] + wlo * xlo0
                    n0_[2 * vv + 1] = n0_[2 * vv + 1] + whi * xhi0
                    n1_[2 * vv] = n1_[2 * vv] + wlo * xlo1
                    n1_[2 * vv + 1] = n1_[2 * vv + 1] + whi * xhi1
                return tuple(n0_), tuple(n1_)

            a0, a1 = lax.fori_loop(0, K, edge, (zero, zero))
            for v in range(VS):
                sl = pl.ds(16 * v, 16)
                o0[g, sl] = a0[v]
                o1[g, sl] = a1[v]
            return carry

        lax.fori_loop(0, G, gloop, 0)
        for c in out_copies(t, p):
            c.start()
        if t + 2 < NCH:
            for c in in_copies(t + 2, p):
                c.start()

    for t in (NCH - 2, NCH - 1):
        for c in out_copies(t, t & 1):
            c.wait()


@functools.cache
def _make_agg_sc():
    return functools.partial(
        pl.kernel,
        mesh=plsc.VectorSubcoreMesh(core_axis_name="c", subcore_axis_name="s"),
        out_type=jax.ShapeDtypeStruct((M, C), jnp.float32),
        scratch_types=[
            pltpu.VMEM((NCH, EPC), jnp.int32),
            pltpu.VMEM((EPC, CW), jnp.int32),
            pltpu.VMEM((EPC, CW), jnp.int32),
            pltpu.VMEM((EPC, 2 * CW), jnp.int32),
            pltpu.VMEM((EPC, 2 * CW), jnp.int32),
            pltpu.VMEM((G, C), jnp.float32),
            pltpu.VMEM((G, C), jnp.float32),
            pltpu.VMEM((G, C), jnp.float32),
            pltpu.VMEM((G, C), jnp.float32),
            pltpu.SemaphoreType.DMA,
            pltpu.SemaphoreType.DMA,
            pltpu.SemaphoreType.DMA,
            pltpu.SemaphoreType.DMA,
        ],
    )(_agg_body)


def _aggregate(xf, psiphi_bf, eidx):
    xi = lax.bitcast_convert_type(
        xf.astype(jnp.bfloat16).reshape(M, CW, 2), jnp.int32)
    xc = jnp.concatenate([xi[:N], xi[N:]], axis=1)
    wi = lax.bitcast_convert_type(psiphi_bf.reshape(NE, CW, 2), jnp.int32)
    return _make_agg_sc()(xc, wi, eidx)


BLK_M = 512
GRID3 = M // BLK_M


def _perm_mat():
    u = np.arange(32)
    dst_blk = np.where(u < 16, 2 * u, 2 * (u - 16) + 1)
    dst = (32 * (np.arange(C) // 32) + dst_blk[np.arange(C) % 32])
    P = np.zeros((C, C), np.float32)
    P[np.arange(C), dst] = 1.0
    return P


_P_NP = _perm_mat()


def _stage3_body(x_ref, agg_ref, sphi_ref, spsi_ref, w1_ref, b1_ref, w2_ref,
                 b2_ref, p_ref, out_ref):
    x = x_ref[...]
    hh = jnp.maximum(
        lax.dot_general(x, w1_ref[...], (((1,), (0,)), ((), ())),
                        precision=lax.Precision.HIGHEST,
                        preferred_element_type=jnp.float32) + b1_ref[...], 0.0)
    pw = lax.dot_general(hh, w2_ref[...], (((1,), (0,)), ((), ())),
                         precision=lax.Precision.HIGHEST,
                         preferred_element_type=jnp.float32) + b2_ref[...]
    agg = lax.dot_general(agg_ref[...], p_ref[...], (((1,), (0,)), ((), ())),
                          precision=lax.Precision.HIGHEST,
                          preferred_element_type=jnp.float32)
    mean_phi = sphi_ref[...] * (1.0 / NE)
    mean_psi = spsi_ref[...] * (1.0 / NE)
    scale = 1.0 / (K * (mean_psi + 1e-06) * (mean_phi + 1e-06))
    out_ref[...] = pw + agg * scale


def _stage3(xf, agg, sphi, spsi, W1, b1, W2, b2):
    full = lambda s: pl.BlockSpec(s, lambda i: (0, 0))
    return pl.pallas_call(
        _stage3_body,
        grid=(GRID3,),
        in_specs=[
            pl.BlockSpec((BLK_M, C), lambda i: (i, 0)),
            pl.BlockSpec((BLK_M, C), lambda i: (i, 0)),
            full((1, C)), full((1, C)),
            full((C, 2 * C)), full((1, 2 * C)), full((2 * C, C)), full((1, C)),
            full((C, C)),
        ],
        out_specs=pl.BlockSpec((BLK_M, C), lambda i: (i, 0)),
        out_shape=jax.ShapeDtypeStruct((M, C), jnp.float32),
    )(xf, agg, sphi, spsi, W1, b1, W2, b2, jnp.asarray(_P_NP))


def kernel(x, W1, b1, W2, b2, pw1, pb1, pw2, pb2, hw1, hb1, hw2, hb2, S_m):
    xf = x.reshape(M, C)
    s_k1 = jnp.concatenate([S_m[1:], S_m[-1:]])
    stab = jnp.stack([S_m, s_k1], axis=0)

    psiphi, sphi, spsi = _stage1(
        jnp.asarray(_XY_NP), jnp.asarray(_CI_NP), jnp.asarray(_R_NP),
        pw1, pb1.reshape(1, C), pw2, pb2.reshape(1, C),
        hw1, hb1.reshape(1, C), hw2.reshape(1, C), hb2.reshape(1, 1), stab)

    agg = _aggregate(xf, psiphi, jnp.asarray(_EIDX_NP))

    out = _stage3(xf, agg, sphi, spsi, W1, b1.reshape(1, 2 * C), W2,
                  b2.reshape(1, C))
    return out.reshape(B, N, C)


# speedup vs baseline: 1.7979x; 1.7979x over previous
import functools

import jax
import jax.numpy as jnp
import numpy as np
from jax import lax
from jax.experimental import pallas as pl
from jax.experimental.pallas import tpu as pltpu
from jax.experimental.pallas import tpu_sc as plsc

B, N, C = 2, 4096, 128
K = 32
NUM_KNOTS = 32
RADIUS = 0.2
NE = N * K
M = B * N


def _graph_jax():
    H = int(N ** 0.5)
    gy, gx = jnp.meshgrid(jnp.linspace(0.0, 1.0, H), jnp.linspace(0.0, 1.0, H),
                          indexing="ij")
    coords = jnp.stack([gy, gx], axis=-1).reshape(N, 2).astype(jnp.float32)
    diff = coords[:, None, :] - coords[None, :, :]
    dist = jnp.linalg.norm(diff, axis=-1)
    dm = jnp.where(dist <= RADIUS, dist, jnp.inf)
    _, jidx = jax.lax.top_k(-dm, K)
    ci = jnp.repeat(coords, K, axis=0)
    cj = coords[jidx.reshape(-1)]
    r = jnp.linalg.norm(ci - cj, axis=-1)
    xy = jnp.concatenate([ci, cj], axis=-1)
    return ci, xy, r, jidx.astype(jnp.int32)


def _build_consts():
    try:
        ci, xy, r, jidx = (np.asarray(a) for a in
                           jax.block_until_ready(jax.jit(_graph_jax)()))
    except Exception:
        H = int(N ** 0.5)
        lin = np.arange(H, dtype=np.float32) * (np.float32(1.0) / np.float32(H - 1))
        gy, gx = np.meshgrid(lin, lin, indexing="ij")
        coords = np.stack([gy, gx], axis=-1).reshape(N, 2).astype(np.float32)
        d = coords[:, None, :] - coords[None, :, :]
        dist = np.sqrt(d[..., 0] * d[..., 0] + d[..., 1] * d[..., 1]).astype(np.float32)
        dm = np.where(dist <= np.float32(RADIUS), dist, np.float32(np.inf))
        jidx = np.argsort(dm, axis=1, kind="stable")[:, :K].astype(np.int32)
        ci = np.repeat(coords, K, axis=0)
        cj = coords[jidx.reshape(-1)]
        dd = ci - cj
        r = np.sqrt(dd[:, 0] * dd[:, 0] + dd[:, 1] * dd[:, 1]).astype(np.float32)
        xy = np.concatenate([ci, cj], axis=-1).astype(np.float32)
    jidx = np.asarray(jidx, dtype=np.int32).reshape(N, K)
    eidx2 = np.ascontiguousarray(jidx.reshape(-1, 4 * K))
    return (np.asarray(ci), np.asarray(xy),
            np.asarray(r, dtype=np.float32).reshape(NE, 1),
            eidx2)


_CI_NP, _XY_NP, _R_NP, _EIDX_NP = _build_consts()


BLK_E = 4096
GRID1 = NE // BLK_E


def _softplus(v):
    return jnp.maximum(v, 0.0) + jnp.log1p(jnp.exp(-jnp.abs(v)))


def _stage1_body(xy_ref, ci_ref, r_ref, pw1_ref, pb1_ref, pw2_ref, pb2_ref,
                 hw1_ref, hb1_ref, hw2t_ref, hb2_ref, stab_ref,
                 psiphi_ref, sphi_ref, spsi_ref):
    xy = xy_ref[...]
    ci = ci_ref[...]
    r = r_ref[...]
    hh = hb1_ref[...] + ci[:, 0:1] * hw1_ref[0:1, :] + ci[:, 1:2] * hw1_ref[1:2, :]
    hh = jnp.maximum(hh, 0.0)
    hlin = jnp.sum(hh * hw2t_ref[...], axis=1, keepdims=True) + hb2_ref[...]
    h = _softplus(hlin)
    rs = jnp.clip(r / (h + 1e-06), 0.0, 1.0)
    idx = jnp.clip(jnp.floor(rs * (NUM_KNOTS - 1)), 0.0, float(NUM_KNOTS - 2))
    idx = idx.astype(jnp.int32)
    step = np.float32(1.0 / (NUM_KNOTS - 1))
    idxf = idx.astype(jnp.float32)
    t_k = idxf * step
    t_k1 = (idxf + 1.0) * step
    wr = (rs - t_k) / (t_k1 - t_k + 1e-08)
    oh = (idx == lax.broadcasted_iota(jnp.int32, (BLK_E, NUM_KNOTS), 1))
    ohf = oh.astype(jnp.float32)
    s_k = jnp.sum(ohf * stab_ref[0:1, :], axis=1, keepdims=True)
    s_k1 = jnp.sum(ohf * stab_ref[1:2, :], axis=1, keepdims=True)
    psi = (1.0 - wr) * s_k + wr * s_k1
    ph = pb1_ref[...]
    for d in range(4):
        ph = ph + xy[:, d:d + 1] * pw1_ref[d:d + 1, :]
    ph = jnp.maximum(ph, 0.0)
    phi = lax.dot_general(ph, pw2_ref[...], (((1,), (0,)), ((), ())),
                          precision=lax.Precision.HIGHEST,
                          preferred_element_type=jnp.float32) + pb2_ref[...]
    u = lax.bitcast_convert_type((psi * phi).astype(jnp.bfloat16), jnp.uint16)
    lo = u[:, :C // 2].astype(jnp.uint32)
    hi = u[:, C // 2:].astype(jnp.uint32)
    psiphi_ref[...] = lax.bitcast_convert_type(lo | (hi << 16), jnp.int32)

    @pl.when(pl.program_id(0) == 0)
    def _():
        sphi_ref[...] = jnp.zeros_like(sphi_ref)
        spsi_ref[...] = jnp.zeros_like(spsi_ref)

    sphi_ref[...] += jnp.sum(jnp.abs(phi), axis=0, keepdims=True)
    spsi_ref[...] += jnp.full((1, C), jnp.sum(jnp.abs(psi)), jnp.float32)


def _stage1(xy, ci, r, pw1, pb1, pw2, pb2, hw1, hb1, hw2t, hb2, stab):
    full = lambda s: pl.BlockSpec(s, lambda i: (0, 0))
    return pl.pallas_call(
        _stage1_body,
        grid=(GRID1,),
        in_specs=[
            pl.BlockSpec((BLK_E, 4), lambda i: (i, 0)),
            pl.BlockSpec((BLK_E, 2), lambda i: (i, 0)),
            pl.BlockSpec((BLK_E, 1), lambda i: (i, 0)),
            full((4, C)), full((1, C)), full((C, C)), full((1, C)),
            full((2, C)), full((1, C)), full((1, C)), full((1, 1)),
            full((2, NUM_KNOTS)),
        ],
        out_specs=[
            pl.BlockSpec((BLK_E, C // 2), lambda i: (i, 0)),
            pl.BlockSpec((1, C), lambda i: (0, 0)),
            pl.BlockSpec((1, C), lambda i: (0, 0)),
        ],
        out_shape=[
            jax.ShapeDtypeStruct((NE, C // 2), jnp.int32),
            jax.ShapeDtypeStruct((1, C), jnp.float32),
            jax.ShapeDtypeStruct((1, C), jnp.float32),
        ],
    )(xy, ci, r, pw1, pb1, pw2, pb2, hw1, hb1, hw2t, hb2, stab)


NW = 32
G = 4
NCORES = 2
VS = C // 16


NPW = N // NW
NCH = NPW // G
EPC = G * K
CW = C // 2
WG = CW // 16


def _agg_body(x_hbm, w_hbm, eidx_hbm, out_hbm, idx_all,
              wb0, wb1, xa0, xa1, ob00, ob01, ob10, ob11,
              semi0, semi1, semo0, semo1):
    wid = lax.axis_index("s") * NCORES + lax.axis_index("c")
    n0 = wid * NPW
    pltpu.sync_copy(eidx_hbm.at[pl.ds(wid * NCH, NCH)], idx_all)

    wbufs = (wb0, wb1)
    xbufs = (xa0, xa1)
    obufs = ((ob00, ob01), (ob10, ob11))
    semis = (semi0, semi1)
    semos = (semo0, semo1)

    def in_copies(t, p):
        ib = n0 + t * G
        return (
            pltpu.make_async_copy(w_hbm.at[pl.ds(ib * K, EPC)], wbufs[p], semis[p]),
            pltpu.make_async_copy(x_hbm.at[idx_all.at[t]], xbufs[p], semis[p]),
        )

    def out_copies(t, p):
        ib = n0 + t * G
        return (
            pltpu.make_async_copy(obufs[p][0], out_hbm.at[pl.ds(ib, G)], semos[p]),
            pltpu.make_async_copy(obufs[p][1], out_hbm.at[pl.ds(N + ib, G)], semos[p]),
        )

    for c in in_copies(0, 0) + in_copies(1, 1):
        c.start()

    zero = tuple(jnp.zeros((16,), jnp.float32) for _ in range(VS))
    for t in range(NCH):
        p = t & 1
        if t >= 2:
            for c in out_copies(t - 2, p):
                c.wait()
        for c in in_copies(t, p):
            c.wait()
        wv_ref, xr = wbufs[p], xbufs[p]
        o0, o1 = obufs[p]

        def gloop(g, carry):
            def unpk(word):
                lo = lax.bitcast_convert_type(lax.shift_left(word, 16), jnp.float32)
                hi = lax.bitcast_convert_type(word & jnp.int32(-65536), jnp.float32)
                return lo, hi

            def edge(k, accs):
                a0, a1 = accs
                e = g * K + k
                n0_, n1_ = list(a0), list(a1)
                for vv in range(WG):
                    sl = pl.ds(16 * vv, 16)
                    wlo, whi = unpk(wv_ref[e, sl])
                    xlo0, xhi0 = unpk(xr[e, sl])
                    xlo1, xhi1 = unpk(xr[e, pl.ds(CW + 16 * vv, 16)])
                    n0_[vv] = n0_[vv] + wlo * xlo0
                    n0_[WG + vv] = n0_[WG + vv] + whi * xhi0
                    n1_[vv] = n1_[vv] + wlo * xlo1
                    n1_[WG + vv] = n1_[WG + vv] + whi * xhi1
                return tuple(n0_), tuple(n1_)

            a0, a1 = lax.fori_loop(0, K, edge, (zero, zero))
            for v in range(VS):
                sl = pl.ds(16 * v, 16)
                o0[g, sl] = a0[v]
                o1[g, sl] = a1[v]
            return carry

        lax.fori_loop(0, G, gloop, 0)
        for c in out_copies(t, p):
            c.start()
        if t + 2 < NCH:
            for c in in_copies(t + 2, p):
                c.start()

    for t in (NCH - 2, NCH - 1):
        for c in out_copies(t, t & 1):
            c.wait()


@functools.cache
def _make_agg_sc():
    return functools.partial(
        pl.kernel,
        mesh=plsc.VectorSubcoreMesh(core_axis_name="c", subcore_axis_name="s"),
        out_type=jax.ShapeDtypeStruct((M, C), jnp.float32),
        scratch_types=[
            pltpu.VMEM((NCH, EPC), jnp.int32),
            pltpu.VMEM((EPC, CW), jnp.int32),
            pltpu.VMEM((EPC, CW), jnp.int32),
            pltpu.VMEM((EPC, 2 * CW), jnp.int32),
            pltpu.VMEM((EPC, 2 * CW), jnp.int32),
            pltpu.VMEM((G, C), jnp.float32),
            pltpu.VMEM((G, C), jnp.float32),
            pltpu.VMEM((G, C), jnp.float32),
            pltpu.VMEM((G, C), jnp.float32),
            pltpu.SemaphoreType.DMA,
            pltpu.SemaphoreType.DMA,
            pltpu.SemaphoreType.DMA,
            pltpu.SemaphoreType.DMA,
        ],
    )(_agg_body)


def _aggregate(xf, psiphi_pk, eidx):
    u = lax.bitcast_convert_type(xf.astype(jnp.bfloat16), jnp.uint16)
    xi = lax.bitcast_convert_type(
        u[:, :CW].astype(jnp.uint32) | (u[:, CW:].astype(jnp.uint32) << 16),
        jnp.int32)
    xc = jnp.concatenate([xi[:N], xi[N:]], axis=1)
    return _make_agg_sc()(xc, psiphi_pk, eidx)


BLK_M = 512
GRID3 = M // BLK_M


def _stage3_body(x_ref, agg_ref, sphi_ref, spsi_ref, w1_ref, b1_ref, w2_ref,
                 b2_ref, out_ref):
    x = x_ref[...]
    hh = jnp.maximum(
        lax.dot_general(x, w1_ref[...], (((1,), (0,)), ((), ())),
                        precision=lax.Precision.HIGHEST,
                        preferred_element_type=jnp.float32) + b1_ref[...], 0.0)
    pw = lax.dot_general(hh, w2_ref[...], (((1,), (0,)), ((), ())),
                         precision=lax.Precision.HIGHEST,
                         preferred_element_type=jnp.float32) + b2_ref[...]
    mean_phi = sphi_ref[...] * (1.0 / NE)
    mean_psi = spsi_ref[...] * (1.0 / NE)
    scale = 1.0 / (K * (mean_psi + 1e-06) * (mean_phi + 1e-06))
    out_ref[...] = pw + agg_ref[...] * scale


def _stage3(xf, agg, sphi, spsi, W1, b1, W2, b2):
    full = lambda s: pl.BlockSpec(s, lambda i: (0, 0))
    return pl.pallas_call(
        _stage3_body,
        grid=(GRID3,),
        in_specs=[
            pl.BlockSpec((BLK_M, C), lambda i: (i, 0)),
            pl.BlockSpec((BLK_M, C), lambda i: (i, 0)),
            full((1, C)), full((1, C)),
            full((C, 2 * C)), full((1, 2 * C)), full((2 * C, C)), full((1, C)),
        ],
        out_specs=pl.BlockSpec((BLK_M, C), lambda i: (i, 0)),
        out_shape=jax.ShapeDtypeStruct((M, C), jnp.float32),
    )(xf, agg, sphi, spsi, W1, b1, W2, b2)


def kernel(x, W1, b1, W2, b2, pw1, pb1, pw2, pb2, hw1, hb1, hw2, hb2, S_m):
    xf = x.reshape(M, C)
    s_k1 = jnp.concatenate([S_m[1:], S_m[-1:]])
    stab = jnp.stack([S_m, s_k1], axis=0)

    psiphi, sphi, spsi = _stage1(
        jnp.asarray(_XY_NP), jnp.asarray(_CI_NP), jnp.asarray(_R_NP),
        pw1, pb1.reshape(1, C), pw2, pb2.reshape(1, C),
        hw1, hb1.reshape(1, C), hw2.reshape(1, C), hb2.reshape(1, 1), stab)

    agg = _aggregate(xf, psiphi, jnp.asarray(_EIDX_NP))

    out = _stage3(xf, agg, sphi, spsi, W1, b1.reshape(1, 2 * C), W2,
                  b2.reshape(1, C))
    return out.reshape(B, N, C)


# speedup vs baseline: 1.8458x; 1.0267x over previous
import functools

import jax
import jax.numpy as jnp
import numpy as np
from jax import lax
from jax.experimental import pallas as pl
from jax.experimental.pallas import tpu as pltpu
from jax.experimental.pallas import tpu_sc as plsc

B, N, C = 2, 4096, 128
K = 32
NUM_KNOTS = 32
RADIUS = 0.2
NE = N * K
M = B * N


def _graph_jax():
    H = int(N ** 0.5)
    gy, gx = jnp.meshgrid(jnp.linspace(0.0, 1.0, H), jnp.linspace(0.0, 1.0, H),
                          indexing="ij")
    coords = jnp.stack([gy, gx], axis=-1).reshape(N, 2).astype(jnp.float32)
    diff = coords[:, None, :] - coords[None, :, :]
    dist = jnp.linalg.norm(diff, axis=-1)
    dm = jnp.where(dist <= RADIUS, dist, jnp.inf)
    _, jidx = jax.lax.top_k(-dm, K)
    ci = jnp.repeat(coords, K, axis=0)
    cj = coords[jidx.reshape(-1)]
    r = jnp.linalg.norm(ci - cj, axis=-1)
    xy = jnp.concatenate([ci, cj], axis=-1)
    return ci, xy, r, jidx.astype(jnp.int32)


def _build_consts():
    try:
        ci, xy, r, jidx = (np.asarray(a) for a in
                           jax.block_until_ready(jax.jit(_graph_jax)()))
    except Exception:
        H = int(N ** 0.5)
        lin = np.arange(H, dtype=np.float32) * (np.float32(1.0) / np.float32(H - 1))
        gy, gx = np.meshgrid(lin, lin, indexing="ij")
        coords = np.stack([gy, gx], axis=-1).reshape(N, 2).astype(np.float32)
        d = coords[:, None, :] - coords[None, :, :]
        dist = np.sqrt(d[..., 0] * d[..., 0] + d[..., 1] * d[..., 1]).astype(np.float32)
        dm = np.where(dist <= np.float32(RADIUS), dist, np.float32(np.inf))
        jidx = np.argsort(dm, axis=1, kind="stable")[:, :K].astype(np.int32)
        ci = np.repeat(coords, K, axis=0)
        cj = coords[jidx.reshape(-1)]
        dd = ci - cj
        r = np.sqrt(dd[:, 0] * dd[:, 0] + dd[:, 1] * dd[:, 1]).astype(np.float32)
        xy = np.concatenate([ci, cj], axis=-1).astype(np.float32)
    jidx = np.asarray(jidx, dtype=np.int32).reshape(N, K)
    eidx2 = np.ascontiguousarray(jidx.reshape(-1, 4 * K))
    return (np.asarray(ci), np.asarray(xy),
            np.asarray(r, dtype=np.float32).reshape(NE, 1),
            eidx2)


_CI_NP, _XY_NP, _R_NP, _EIDX_NP = _build_consts()


BLK_E = 4096
GRID1 = NE // BLK_E


def _softplus(v):
    return jnp.maximum(v, 0.0) + jnp.log1p(jnp.exp(-jnp.abs(v)))


def _stage1_body(xy_ref, ci_ref, r_ref, pw1_ref, pb1_ref, pw2_ref, pb2_ref,
                 hw1_ref, hb1_ref, hw2t_ref, hb2_ref, stab_ref,
                 psiphi_ref, sphi_ref, spsi_ref):
    xy = xy_ref[...]
    ci = ci_ref[...]
    r = r_ref[...]
    hh = hb1_ref[...] + ci[:, 0:1] * hw1_ref[0:1, :] + ci[:, 1:2] * hw1_ref[1:2, :]
    hh = jnp.maximum(hh, 0.0)
    hlin = jnp.sum(hh * hw2t_ref[...], axis=1, keepdims=True) + hb2_ref[...]
    h = _softplus(hlin)
    rs = jnp.clip(r / (h + 1e-06), 0.0, 1.0)
    idx = jnp.clip(jnp.floor(rs * (NUM_KNOTS - 1)), 0.0, float(NUM_KNOTS - 2))
    idx = idx.astype(jnp.int32)
    step = np.float32(1.0 / (NUM_KNOTS - 1))
    idxf = idx.astype(jnp.float32)
    t_k = idxf * step
    t_k1 = (idxf + 1.0) * step
    wr = (rs - t_k) / (t_k1 - t_k + 1e-08)
    oh = (idx == lax.broadcasted_iota(jnp.int32, (BLK_E, NUM_KNOTS), 1))
    ohf = oh.astype(jnp.float32)
    s_k = jnp.sum(ohf * stab_ref[0:1, :], axis=1, keepdims=True)
    s_k1 = jnp.sum(ohf * stab_ref[1:2, :], axis=1, keepdims=True)
    psi = (1.0 - wr) * s_k + wr * s_k1
    ph = pb1_ref[...]
    for d in range(4):
        ph = ph + xy[:, d:d + 1] * pw1_ref[d:d + 1, :]
    ph = jnp.maximum(ph, 0.0)
    phi = lax.dot_general(ph, pw2_ref[...], (((1,), (0,)), ((), ())),
                          precision=lax.Precision.HIGHEST,
                          preferred_element_type=jnp.float32) + pb2_ref[...]
    psiphi_ref[...] = psi * phi

    @pl.when(pl.program_id(0) == 0)
    def _():
        sphi_ref[...] = jnp.zeros_like(sphi_ref)
        spsi_ref[...] = jnp.zeros_like(spsi_ref)

    sphi_ref[...] += jnp.sum(jnp.abs(phi), axis=0, keepdims=True)
    spsi_ref[...] += jnp.full((1, C), jnp.sum(jnp.abs(psi)), jnp.float32)


def _stage1(xy, ci, r, pw1, pb1, pw2, pb2, hw1, hb1, hw2t, hb2, stab):
    full = lambda s: pl.BlockSpec(s, lambda i: (0, 0))
    return pl.pallas_call(
        _stage1_body,
        grid=(GRID1,),
        in_specs=[
            pl.BlockSpec((BLK_E, 4), lambda i: (i, 0)),
            pl.BlockSpec((BLK_E, 2), lambda i: (i, 0)),
            pl.BlockSpec((BLK_E, 1), lambda i: (i, 0)),
            full((4, C)), full((1, C)), full((C, C)), full((1, C)),
            full((2, C)), full((1, C)), full((1, C)), full((1, 1)),
            full((2, NUM_KNOTS)),
        ],
        out_specs=[
            pl.BlockSpec((BLK_E, C), lambda i: (i, 0)),
            pl.BlockSpec((1, C), lambda i: (0, 0)),
            pl.BlockSpec((1, C), lambda i: (0, 0)),
        ],
        out_shape=[
            jax.ShapeDtypeStruct((NE, C), jnp.float32),
            jax.ShapeDtypeStruct((1, C), jnp.float32),
            jax.ShapeDtypeStruct((1, C), jnp.float32),
        ],
    )(xy, ci, r, pw1, pb1, pw2, pb2, hw1, hb1, hw2t, hb2, stab)


NW = 32
G = 4
NCORES = 2
VS = C // 16


NPW = N // NW
NCH = NPW // G
EPC = G * K
CW = C // 2
WG = CW // 16


def _agg_body(x_hbm, w_hbm, eidx_hbm, out_hbm, idx_all,
              wb0, wb1, xa0, xa1, ob00, ob01, ob10, ob11,
              semi0, semi1, semo0, semo1):
    wid = lax.axis_index("s") * NCORES + lax.axis_index("c")
    n0 = wid * NPW
    pltpu.sync_copy(eidx_hbm.at[pl.ds(wid * NCH, NCH)], idx_all)

    wbufs = (wb0, wb1)
    xbufs = (xa0, xa1)
    obufs = ((ob00, ob01), (ob10, ob11))
    semis = (semi0, semi1)
    semos = (semo0, semo1)

    def in_copies(t, p):
        ib = n0 + t * G
        return (
            pltpu.make_async_copy(w_hbm.at[pl.ds(ib * K, EPC)], wbufs[p], semis[p]),
            pltpu.make_async_copy(x_hbm.at[idx_all.at[t]], xbufs[p], semis[p]),
        )

    def out_copies(t, p):
        ib = n0 + t * G
        return (
            pltpu.make_async_copy(obufs[p][0], out_hbm.at[pl.ds(ib, G)], semos[p]),
            pltpu.make_async_copy(obufs[p][1], out_hbm.at[pl.ds(N + ib, G)], semos[p]),
        )

    for c in in_copies(0, 0) + in_copies(1, 1):
        c.start()

    zero = tuple(jnp.zeros((16,), jnp.float32) for _ in range(VS))
    for t in range(NCH):
        p = t & 1
        if t >= 2:
            for c in out_copies(t - 2, p):
                c.wait()
        for c in in_copies(t, p):
            c.wait()
        wv_ref, xr = wbufs[p], xbufs[p]
        o0, o1 = obufs[p]

        def gloop(g, carry):
            def unpk(word):
                lo = lax.bitcast_convert_type(lax.shift_left(word, 16), jnp.float32)
                hi = lax.bitcast_convert_type(word & jnp.int32(-65536), jnp.float32)
                return lo, hi

            def edge(k, accs):
                a0, a1 = accs
                e = g * K + k
                n0_, n1_ = list(a0), list(a1)
                for vv in range(WG):
                    sl = pl.ds(16 * vv, 16)
                    wlo = wv_ref[e, sl]
                    whi = wv_ref[e, pl.ds(CW + 16 * vv, 16)]
                    xlo0, xhi0 = unpk(xr[e, sl])
                    xlo1, xhi1 = unpk(xr[e, pl.ds(CW + 16 * vv, 16)])
                    n0_[vv] = n0_[vv] + wlo * xlo0
                    n0_[WG + vv] = n0_[WG + vv] + whi * xhi0
                    n1_[vv] = n1_[vv] + wlo * xlo1
                    n1_[WG + vv] = n1_[WG + vv] + whi * xhi1
                return tuple(n0_), tuple(n1_)

            a0, a1 = lax.fori_loop(0, K, edge, (zero, zero))
            for v in range(VS):
                sl = pl.ds(16 * v, 16)
                o0[g, sl] = a0[v]
                o1[g, sl] = a1[v]
            return carry

        lax.fori_loop(0, G, gloop, 0)
        for c in out_copies(t, p):
            c.start()
        if t + 2 < NCH:
            for c in in_copies(t + 2, p):
                c.start()

    for t in (NCH - 2, NCH - 1):
        for c in out_copies(t, t & 1):
            c.wait()


@functools.cache
def _make_agg_sc():
    return functools.partial(
        pl.kernel,
        mesh=plsc.VectorSubcoreMesh(core_axis_name="c", subcore_axis_name="s"),
        out_type=jax.ShapeDtypeStruct((M, C), jnp.float32),
        scratch_types=[
            pltpu.VMEM((NCH, EPC), jnp.int32),
            pltpu.VMEM((EPC, C), jnp.float32),
            pltpu.VMEM((EPC, C), jnp.float32),
            pltpu.VMEM((EPC, 2 * CW), jnp.int32),
            pltpu.VMEM((EPC, 2 * CW), jnp.int32),
            pltpu.VMEM((G, C), jnp.float32),
            pltpu.VMEM((G, C), jnp.float32),
            pltpu.VMEM((G, C), jnp.float32),
            pltpu.VMEM((G, C), jnp.float32),
            pltpu.SemaphoreType.DMA,
            pltpu.SemaphoreType.DMA,
            pltpu.SemaphoreType.DMA,
            pltpu.SemaphoreType.DMA,
        ],
    )(_agg_body)


def _aggregate(xf, psiphi, eidx):
    u = lax.bitcast_convert_type(xf.astype(jnp.bfloat16), jnp.uint16)
    xi = lax.bitcast_convert_type(
        u[:, :CW].astype(jnp.uint32) | (u[:, CW:].astype(jnp.uint32) << 16),
        jnp.int32)
    xc = jnp.concatenate([xi[:N], xi[N:]], axis=1)
    return _make_agg_sc()(xc, psiphi, eidx)


BLK_M = 512
GRID3 = M // BLK_M


def _stage3_body(x_ref, agg_ref, sphi_ref, spsi_ref, w1_ref, b1_ref, w2_ref,
                 b2_ref, out_ref):
    x = x_ref[...]
    hh = jnp.maximum(
        lax.dot_general(x, w1_ref[...], (((1,), (0,)), ((), ())),
                        precision=lax.Precision.HIGHEST,
                        preferred_element_type=jnp.float32) + b1_ref[...], 0.0)
    pw = lax.dot_general(hh, w2_ref[...], (((1,), (0,)), ((), ())),
                         precision=lax.Precision.HIGHEST,
                         preferred_element_type=jnp.float32) + b2_ref[...]
    mean_phi = sphi_ref[...] * (1.0 / NE)
    mean_psi = spsi_ref[...] * (1.0 / NE)
    scale = 1.0 / (K * (mean_psi + 1e-06) * (mean_phi + 1e-06))
    out_ref[...] = pw + agg_ref[...] * scale


def _stage3(xf, agg, sphi, spsi, W1, b1, W2, b2):
    full = lambda s: pl.BlockSpec(s, lambda i: (0, 0))
    return pl.pallas_call(
        _stage3_body,
        grid=(GRID3,),
        in_specs=[
            pl.BlockSpec((BLK_M, C), lambda i: (i, 0)),
            pl.BlockSpec((BLK_M, C), lambda i: (i, 0)),
            full((1, C)), full((1, C)),
            full((C, 2 * C)), full((1, 2 * C)), full((2 * C, C)), full((1, C)),
        ],
        out_specs=pl.BlockSpec((BLK_M, C), lambda i: (i, 0)),
        out_shape=jax.ShapeDtypeStruct((M, C), jnp.float32),
    )(xf, agg, sphi, spsi, W1, b1, W2, b2)


def kernel(x, W1, b1, W2, b2, pw1, pb1, pw2, pb2, hw1, hb1, hw2, hb2, S_m):
    xf = x.reshape(M, C)
    s_k1 = jnp.concatenate([S_m[1:], S_m[-1:]])
    stab = jnp.stack([S_m, s_k1], axis=0)

    psiphi, sphi, spsi = _stage1(
        jnp.asarray(_XY_NP), jnp.asarray(_CI_NP), jnp.asarray(_R_NP),
        pw1, pb1.reshape(1, C), pw2, pb2.reshape(1, C),
        hw1, hb1.reshape(1, C), hw2.reshape(1, C), hb2.reshape(1, 1), stab)

    agg = _aggregate(xf, psiphi, jnp.asarray(_EIDX_NP))

    out = _stage3(xf, agg, sphi, spsi, W1, b1.reshape(1, 2 * C), W2,
                  b2.reshape(1, C))
    return out.reshape(B, N, C)
